# baseline (speedup 1.0000x reference)
import jax
import jax.numpy as jnp
from jax.experimental import pallas as pl
from jax.experimental.pallas import tpu as pltpu

N = 1024
D_EDGE = 4
BR = 128
NB = N // BR


def _chebnet_kernel(a0_ref, a1_ref, a2_ref, a3_ref, x_ref,
                    w1_ref, b1_ref, w2_ref, b2_ref,
                    out_ref, a01_scr, deg_scr):
    i = pl.program_id(0)

    @pl.when(i < NB)
    def _build_block():
        m = jnp.maximum(jnp.maximum(a0_ref[...], a1_ref[...]),
                        jnp.maximum(a2_ref[...], a3_ref[...]))
        valid = m.astype(jnp.float32) != 0.0
        rows = jax.lax.broadcasted_iota(jnp.int32, (BR, N), 0) + i * BR
        cols = jax.lax.broadcasted_iota(jnp.int32, (BR, N), 1)
        w = jnp.where(valid & (rows != cols), 1.0, 0.0)
        a01_scr[pl.ds(i * BR, BR), :] = w.astype(jnp.bfloat16)
        deg_scr[pl.ds(i * BR, BR), :] = jnp.sum(w, axis=1, keepdims=True)

    @pl.when(i == NB)
    def _compute():
        deg = deg_scr[...]
        dis = jnp.where(deg > 0.0, jax.lax.rsqrt(deg), 0.0)
        a01 = a01_scr[...]
        x = x_ref[...]

        def smul(v):
            vb = (dis * v).astype(jnp.bfloat16)
            return -dis * jnp.dot(a01, vb, preferred_element_type=jnp.float32)

        def cheb(v, w_ref, b_ref):
            t1 = smul(v)
            t2 = 2.0 * smul(t1) - v
            o = (jnp.dot(v, w_ref[0], preferred_element_type=jnp.float32)
                 + jnp.dot(t1, w_ref[1], preferred_element_type=jnp.float32)
                 + jnp.dot(t2, w_ref[2], preferred_element_type=jnp.float32))
            return o + b_ref[...]

        h = jnp.maximum(cheb(x, w1_ref, b1_ref), 0.0)
        o = cheb(h, w2_ref, b2_ref)
        m = jnp.max(o, axis=1, keepdims=True)
        e = jnp.exp(o - m)
        out_ref[...] = e / jnp.sum(e, axis=1, keepdims=True)


def kernel(feat_matrix, adj_matrix, get_item_index, set_index, val_index,
           mask_matrix, W1, b1, W2, b2):
    n, f0 = feat_matrix.shape
    f1 = W1.shape[-1]
    f2 = W2.shape[-1]
    ab = adj_matrix.astype(jnp.bfloat16)
    chans = [ab[:, :, e] for e in range(D_EDGE)]
    b1r = b1.reshape(1, f1)
    b2r = b2.reshape(1, f2)

    adj_spec = pl.BlockSpec((BR, n), lambda i: (jnp.minimum(i, NB - 1), 0))
    out = pl.pallas_call(
        _chebnet_kernel,
        grid=(NB + 1,),
        in_specs=[
            adj_spec, adj_spec, adj_spec, adj_spec,
            pl.BlockSpec((n, f0), lambda i: (0, 0)),
            pl.BlockSpec((W1.shape[0], f0, f1), lambda i: (0, 0, 0)),
            pl.BlockSpec((1, f1), lambda i: (0, 0)),
            pl.BlockSpec((W2.shape[0], f1, f2), lambda i: (0, 0, 0)),
            pl.BlockSpec((1, f2), lambda i: (0, 0)),
        ],
        out_specs=pl.BlockSpec((n, f2), lambda i: (0, 0)),
        out_shape=jax.ShapeDtypeStruct((n, f2), jnp.float32),
        scratch_shapes=[
            pltpu.VMEM((n, n), jnp.bfloat16),
            pltpu.VMEM((n, 1), jnp.float32),
        ],
        compiler_params=pltpu.CompilerParams(
            dimension_semantics=("arbitrary",),
        ),
    )(*chans, feat_matrix, W1, b1r, W2, b2r)
    return out
